# baseline (speedup 1.0000x reference)
import functools

import jax
import jax.numpy as jnp
from jax import lax
from jax.experimental import pallas as pl
from jax.experimental.pallas import tpu as pltpu
from jax.experimental.pallas import tpu_sc as plsc

N_CORES = 2
N_SUBCORES = 16
NW = N_CORES * N_SUBCORES
CH = 112
LANES = 16
NBUF = 3
NSLOT = 6
UNROLL = 6


def _sc_segment_sum(queue, idx2, w, n_chunks):
    n_nodes, d_feat = queue.shape
    rows_per_tile = 624
    tail_start = rows_per_tile * N_SUBCORES
    tail_rows = n_nodes - tail_start
    d_vregs = d_feat // LANES
    n_groups = CH // LANES

    assert n_chunks % UNROLL == 0
    n_iters = n_chunks // UNROLL

    mesh = plsc.VectorSubcoreMesh(core_axis_name="c", subcore_axis_name="s")

    @functools.partial(
        pl.kernel,
        out_type=jax.ShapeDtypeStruct((N_CORES, n_nodes, d_feat), jnp.float32),
        mesh=mesh,
        compiler_params=pltpu.CompilerParams(needs_layout_passes=False, use_tc_tiling_on_sc=True),
        scratch_types=[
            pltpu.VMEM_SHARED((n_nodes, d_feat), jnp.float32),
        ]
        + [pltpu.VMEM((CH, d_feat), jnp.float32) for _ in range(NBUF)]
        + [pltpu.VMEM((2, CH), jnp.int32) for _ in range(NSLOT)]
        + [pltpu.VMEM((CH,), jnp.float32) for _ in range(NSLOT)]
        + [pltpu.SemaphoreType.DMA for _ in range(2 * NBUF + NSLOT)],
    )
    def k(queue_hbm, idx2_hbm, w_hbm, out_hbm, acc, *refs):
        rows = refs[:NBUF]
        idx_v = refs[NBUF:NBUF + NSLOT]
        w_v = refs[NBUF + NSLOT:NBUF + 2 * NSLOT]
        sems = refs[NBUF + 2 * NSLOT:]
        gsem = sems[:NBUF]
        ssem = sems[NBUF:2 * NBUF]
        isem = sems[2 * NBUF:]
        c = lax.axis_index("c")
        s = lax.axis_index("s")
        tile = c * N_SUBCORES + s

        def start_idx(g, slot):
            pltpu.async_copy(idx2_hbm.at[tile, g], idx_v[slot], isem[slot])
            pltpu.async_copy(w_hbm.at[tile, g], w_v[slot], isem[slot])

        def wait_idx(g, slot):
            pltpu.make_async_copy(idx2_hbm.at[tile, g], idx_v[slot],
                                  isem[slot]).wait()
            pltpu.make_async_copy(w_hbm.at[tile, g], w_v[slot],
                                  isem[slot]).wait()

        def start_gather(slot, b):
            pltpu.async_copy(queue_hbm.at[idx_v[slot].at[0]], rows[b],
                             gsem[b])

        def wait_gather(b):
            pltpu.make_async_copy(queue_hbm.at[idx_v[0].at[0]], rows[b],
                                  gsem[b]).wait()

        def drain_scatter(b):
            pltpu.make_async_copy(rows[b], acc.at[idx_v[0].at[1]],
                                  ssem[b]).wait()

        def zrow(i, carry):
            for kk in range(d_vregs):
                rows[0][i, pl.ds(kk * LANES, LANES)] = jnp.zeros(
                    (LANES,), jnp.float32)
            return carry
        lax.fori_loop(0, CH, zrow, 0)
        row0 = s * rows_per_tile
        for off, cnt in ((0, 112), (112, 112), (224, 112), (336, 112),
                         (448, 112), (560, 64)):
            pltpu.sync_copy(rows[0].at[pl.ds(0, cnt)],
                            acc.at[pl.ds(row0 + off, cnt)])

        @pl.when(s == N_SUBCORES - 1)
        def _zero_tail():
            pltpu.sync_copy(rows[0].at[pl.ds(0, tail_rows)],
                            acc.at[pl.ds(tail_start, tail_rows)])

        start_idx(0, 0)
        start_idx(1, 1)
        wait_idx(0, 0)
        start_gather(0, 0)

        plsc.subcore_barrier()

        def ring(i, carry):
            g0 = UNROLL * i
            for u in range(UNROLL):
                g = g0 + u
                b = u % NBUF
                wait_gather(b)

                if u >= 2:
                    drain_scatter((u + 1) % NBUF)
                else:
                    @pl.when(i > 0)
                    def _():
                        drain_scatter((u + 1) % NBUF)

                def _gather_next():
                    wait_idx(g + 1, (u + 1) % NSLOT)
                    start_gather((u + 1) % NSLOT, (u + 1) % NBUF)
                if u == UNROLL - 1:
                    @pl.when(g + 1 < n_chunks)
                    def _():
                        _gather_next()
                else:
                    _gather_next()

                if u >= UNROLL - 2:
                    @pl.when(g + 2 < n_chunks)
                    def _():
                        start_idx(g + 2, (u + 2) % NSLOT)
                else:
                    start_idx(g + 2, (u + 2) % NSLOT)

                @plsc.parallel_loop(0, n_groups, unroll=2)
                def _scale(gr):
                    wvec = w_v[u][pl.ds(gr * LANES, LANES)]
                    for j in range(LANES):
                        wb = lax.gather(
                            wvec,
                            jnp.full((LANES, 1), j, jnp.int32),
                            lax.GatherDimensionNumbers(
                                offset_dims=(),
                                collapsed_slice_dims=(0,),
                                start_index_map=(0,)),
                            (1,),
                            mode=lax.GatherScatterMode.PROMISE_IN_BOUNDS)
                        re = rows[b].at[gr * LANES + j]
                        for kk in range(d_vregs):
                            seg = re[pl.ds(kk * LANES, LANES)]
                            re[pl.ds(kk * LANES, LANES)] = seg * wb

                pltpu.async_copy(rows[b], acc.at[idx_v[u].at[1]], ssem[b],
                                 add=True)
            return carry
        lax.fori_loop(0, n_iters, ring, 0)

        drain_scatter((n_chunks - 2) % NBUF)
        drain_scatter((n_chunks - 1) % NBUF)
        plsc.subcore_barrier()

        pltpu.sync_copy(acc.at[pl.ds(row0, rows_per_tile)],
                        out_hbm.at[c, pl.ds(row0, rows_per_tile)])

        @pl.when(s == N_SUBCORES - 1)
        def _copy_tail():
            pltpu.sync_copy(acc.at[pl.ds(tail_start, tail_rows)],
                            out_hbm.at[c, pl.ds(tail_start, tail_rows)])

    return k(queue, idx2, w)


def _tc_combine(partials):
    _, n_nodes, d_feat = partials.shape
    blk = 1000

    def add_body(p_ref, o_ref):
        o_ref[...] = p_ref[0] + p_ref[1]

    return pl.pallas_call(
        add_body,
        grid=(n_nodes // blk,),
        in_specs=[pl.BlockSpec((N_CORES, blk, d_feat), lambda i: (0, i, 0))],
        out_specs=pl.BlockSpec((blk, d_feat), lambda i: (i, 0)),
        out_shape=jax.ShapeDtypeStruct((n_nodes, d_feat), jnp.float32),
    )(partials)


def kernel(queue, edge_index, edge_weight):
    n_nodes = queue.shape[0]
    src = edge_index[0].astype(jnp.int32)
    dst = edge_index[1].astype(jnp.int32)
    w = edge_weight[:, 0]

    n_edges = src.shape[0]
    quantum = NW * CH * UNROLL
    e_pad = -(-n_edges // quantum) * quantum
    n_chunks = e_pad // (NW * CH)
    pad = e_pad - n_edges
    if pad:
        pad_idx = jnp.arange(pad, dtype=jnp.int32) % n_nodes
        src = jnp.concatenate([src, pad_idx])
        dst = jnp.concatenate([dst, pad_idx])
        w = jnp.concatenate([w, jnp.zeros((pad,), jnp.float32)])

    idx2 = jnp.stack([src.reshape(NW, n_chunks, CH),
                      dst.reshape(NW, n_chunks, CH)], axis=2)
    w = w.reshape(NW, n_chunks, CH)

    partials = _sc_segment_sum(queue, idx2, w, n_chunks)
    return _tc_combine(partials)


# speedup vs baseline: 1.0677x; 1.0677x over previous
import functools

import jax
import jax.numpy as jnp
from jax import lax
from jax.experimental import pallas as pl
from jax.experimental.pallas import tpu as pltpu
from jax.experimental.pallas import tpu_sc as plsc

N_CORES = 2
N_SUBCORES = 16
NW = N_CORES * N_SUBCORES
CH = 112
LANES = 16
NBUF = 3
NSLOT = 6
UNROLL = 6


def _sc_segment_sum(queue, src, dst, w, n_chunks):
    n_nodes, d_feat = queue.shape
    rows_per_tile = 624
    tail_start = rows_per_tile * N_SUBCORES
    tail_rows = n_nodes - tail_start
    d_vregs = d_feat // LANES
    n_groups = CH // LANES

    assert n_chunks % UNROLL == 0
    n_iters = n_chunks // UNROLL

    mesh = plsc.VectorSubcoreMesh(core_axis_name="c", subcore_axis_name="s")

    @functools.partial(
        pl.kernel,
        out_type=jax.ShapeDtypeStruct((N_CORES, n_nodes, d_feat), jnp.float32),
        mesh=mesh,
        compiler_params=pltpu.CompilerParams(needs_layout_passes=False),
        scratch_types=[
            pltpu.VMEM_SHARED((n_nodes, d_feat), jnp.float32),
        ]
        + [pltpu.VMEM((CH, d_feat), jnp.float32) for _ in range(NBUF)]
        + [pltpu.VMEM((CH,), jnp.int32) for _ in range(2 * NSLOT)]
        + [pltpu.VMEM((CH,), jnp.float32) for _ in range(NSLOT)]
        + [pltpu.SemaphoreType.DMA for _ in range(2 * NBUF + NSLOT)],
    )
    def k(queue_hbm, src_hbm, dst_hbm, w_hbm, out_hbm, acc, *refs):
        rows = refs[:NBUF]
        src_v = refs[NBUF:NBUF + NSLOT]
        dst_v = refs[NBUF + NSLOT:NBUF + 2 * NSLOT]
        w_v = refs[NBUF + 2 * NSLOT:NBUF + 3 * NSLOT]
        sems = refs[NBUF + 3 * NSLOT:]
        gsem = sems[:NBUF]
        ssem = sems[NBUF:2 * NBUF]
        isem = sems[2 * NBUF:]
        c = lax.axis_index("c")
        s = lax.axis_index("s")
        tile = c * N_SUBCORES + s

        def start_idx(g, slot):
            base = (tile * n_chunks + g) * CH
            pltpu.async_copy(src_hbm.at[pl.ds(base, CH)], src_v[slot],
                             isem[slot])
            pltpu.async_copy(dst_hbm.at[pl.ds(base, CH)], dst_v[slot],
                             isem[slot])
            pltpu.async_copy(w_hbm.at[pl.ds(base, CH)], w_v[slot],
                             isem[slot])

        def wait_idx(g, slot):
            base = (tile * n_chunks + g) * CH
            pltpu.make_async_copy(src_hbm.at[pl.ds(base, CH)], src_v[slot],
                                  isem[slot]).wait()
            pltpu.make_async_copy(dst_hbm.at[pl.ds(base, CH)], dst_v[slot],
                                  isem[slot]).wait()
            pltpu.make_async_copy(w_hbm.at[pl.ds(base, CH)], w_v[slot],
                                  isem[slot]).wait()

        def start_gather(slot, b):
            pltpu.async_copy(queue_hbm.at[src_v[slot]], rows[b], gsem[b])

        def wait_gather(b):
            pltpu.make_async_copy(queue_hbm.at[src_v[0]], rows[b],
                                  gsem[b]).wait()

        def drain_scatter(b):
            pltpu.make_async_copy(rows[b], acc.at[dst_v[0]],
                                  ssem[b]).wait()

        def zrow(i, carry):
            for kk in range(d_vregs):
                rows[0][i, pl.ds(kk * LANES, LANES)] = jnp.zeros(
                    (LANES,), jnp.float32)
            return carry
        lax.fori_loop(0, CH, zrow, 0)
        row0 = s * rows_per_tile
        for off, cnt in ((0, 112), (112, 112), (224, 112), (336, 112),
                         (448, 112), (560, 64)):
            pltpu.sync_copy(rows[0].at[pl.ds(0, cnt)],
                            acc.at[pl.ds(row0 + off, cnt)])

        @pl.when(s == N_SUBCORES - 1)
        def _zero_tail():
            pltpu.sync_copy(rows[0].at[pl.ds(0, tail_rows)],
                            acc.at[pl.ds(tail_start, tail_rows)])

        start_idx(0, 0)
        start_idx(1, 1)
        wait_idx(0, 0)
        start_gather(0, 0)

        plsc.subcore_barrier()

        def ring(i, carry):
            g0 = UNROLL * i
            for u in range(UNROLL):
                g = g0 + u
                b = u % NBUF
                wait_gather(b)

                if u >= 2:
                    drain_scatter((u + 1) % NBUF)
                else:
                    @pl.when(i > 0)
                    def _():
                        drain_scatter((u + 1) % NBUF)

                def _gather_next():
                    wait_idx(g + 1, (u + 1) % NSLOT)
                    start_gather((u + 1) % NSLOT, (u + 1) % NBUF)
                if u == UNROLL - 1:
                    @pl.when(g + 1 < n_chunks)
                    def _():
                        _gather_next()
                else:
                    _gather_next()

                if u >= UNROLL - 2:
                    @pl.when(g + 2 < n_chunks)
                    def _():
                        start_idx(g + 2, (u + 2) % NSLOT)
                else:
                    start_idx(g + 2, (u + 2) % NSLOT)

                @plsc.parallel_loop(0, n_groups, unroll=2)
                def _scale(gr):
                    wvec = w_v[u][pl.ds(gr * LANES, LANES)]
                    for j in range(LANES):
                        wb = lax.gather(
                            wvec,
                            jnp.full((LANES, 1), j, jnp.int32),
                            lax.GatherDimensionNumbers(
                                offset_dims=(),
                                collapsed_slice_dims=(0,),
                                start_index_map=(0,)),
                            (1,),
                            mode=lax.GatherScatterMode.PROMISE_IN_BOUNDS)
                        re = rows[b].at[gr * LANES + j]
                        for kk in range(d_vregs):
                            seg = re[pl.ds(kk * LANES, LANES)]
                            re[pl.ds(kk * LANES, LANES)] = seg * wb

                pltpu.async_copy(rows[b], acc.at[dst_v[u]], ssem[b],
                                 add=True)
            return carry
        lax.fori_loop(0, n_iters, ring, 0)

        drain_scatter((n_chunks - 2) % NBUF)
        drain_scatter((n_chunks - 1) % NBUF)
        plsc.subcore_barrier()

        pltpu.sync_copy(acc.at[pl.ds(row0, rows_per_tile)],
                        out_hbm.at[c, pl.ds(row0, rows_per_tile)])

        @pl.when(s == N_SUBCORES - 1)
        def _copy_tail():
            pltpu.sync_copy(acc.at[pl.ds(tail_start, tail_rows)],
                            out_hbm.at[c, pl.ds(tail_start, tail_rows)])

    return k(queue, src, dst, w)


def _tc_combine(partials):
    _, n_nodes, d_feat = partials.shape
    blk = 1000

    def add_body(p_ref, o_ref):
        o_ref[...] = p_ref[0] + p_ref[1]

    return pl.pallas_call(
        add_body,
        grid=(n_nodes // blk,),
        in_specs=[pl.BlockSpec((N_CORES, blk, d_feat), lambda i: (0, i, 0))],
        out_specs=pl.BlockSpec((blk, d_feat), lambda i: (i, 0)),
        out_shape=jax.ShapeDtypeStruct((n_nodes, d_feat), jnp.float32),
    )(partials)


def kernel(queue, edge_index, edge_weight):
    n_nodes = queue.shape[0]
    src = edge_index[0].astype(jnp.int32)
    dst = edge_index[1].astype(jnp.int32)
    w = edge_weight[:, 0]

    n_edges = src.shape[0]
    quantum = NW * CH * UNROLL
    e_pad = -(-n_edges // quantum) * quantum
    n_chunks = e_pad // (NW * CH)
    pad = e_pad - n_edges
    if pad:
        pad_idx = jnp.arange(pad, dtype=jnp.int32) % n_nodes
        src = jnp.concatenate([src, pad_idx])
        dst = jnp.concatenate([dst, pad_idx])
        w = jnp.concatenate([w, jnp.zeros((pad,), jnp.float32)])

    partials = _sc_segment_sum(queue, src, dst, w, n_chunks)
    return _tc_combine(partials)


# speedup vs baseline: 1.1666x; 1.0927x over previous
import functools

import jax
import jax.numpy as jnp
from jax import lax
from jax.experimental import pallas as pl
from jax.experimental.pallas import tpu as pltpu
from jax.experimental.pallas import tpu_sc as plsc

N_CORES = 2
N_SUBCORES = 16
NW = N_CORES * N_SUBCORES
CH = 112
LANES = 16
NBUF = 3
NSLOT = 6
UNROLL = 6


def _sc_segment_sum(queue, src, dst, w, n_chunks):
    n_nodes, d_feat = queue.shape
    rows_per_tile = 624
    tail_start = rows_per_tile * N_SUBCORES
    tail_rows = n_nodes - tail_start
    d_vregs = d_feat // LANES
    n_groups = CH // LANES

    assert n_chunks % UNROLL == 0
    n_iters = n_chunks // UNROLL

    mesh = plsc.VectorSubcoreMesh(core_axis_name="c", subcore_axis_name="s")

    @functools.partial(
        pl.kernel,
        out_type=jax.ShapeDtypeStruct((N_CORES, n_nodes, d_feat), jnp.float32),
        mesh=mesh,
        compiler_params=pltpu.CompilerParams(needs_layout_passes=False),
        scratch_types=[
            pltpu.VMEM_SHARED((n_nodes, d_feat), jnp.float32),
        ]
        + [pltpu.VMEM((CH, d_feat), jnp.float32) for _ in range(NBUF)]
        + [pltpu.VMEM((CH,), jnp.int32) for _ in range(2 * NSLOT)]
        + [pltpu.VMEM((CH,), jnp.float32) for _ in range(NSLOT)]
        + [pltpu.SemaphoreType.DMA for _ in range(2 * NBUF + NSLOT)],
    )
    def k(queue_hbm, src_hbm, dst_hbm, w_hbm, out_hbm, acc, *refs):
        rows = refs[:NBUF]
        src_v = refs[NBUF:NBUF + NSLOT]
        dst_v = refs[NBUF + NSLOT:NBUF + 2 * NSLOT]
        w_v = refs[NBUF + 2 * NSLOT:NBUF + 3 * NSLOT]
        sems = refs[NBUF + 3 * NSLOT:]
        gsem = sems[:NBUF]
        ssem = sems[NBUF:2 * NBUF]
        isem = sems[2 * NBUF:]
        c = lax.axis_index("c")
        s = lax.axis_index("s")
        tile = c * N_SUBCORES + s

        def start_idx(g, slot):
            base = (tile * n_chunks + g) * CH
            pltpu.async_copy(src_hbm.at[pl.ds(base, CH)], src_v[slot],
                             isem[slot])
            pltpu.async_copy(dst_hbm.at[pl.ds(base, CH)], dst_v[slot],
                             isem[slot])
            pltpu.async_copy(w_hbm.at[pl.ds(base, CH)], w_v[slot],
                             isem[slot])

        def wait_idx(g, slot):
            base = (tile * n_chunks + g) * CH
            pltpu.make_async_copy(src_hbm.at[pl.ds(base, CH)], src_v[slot],
                                  isem[slot]).wait()
            pltpu.make_async_copy(dst_hbm.at[pl.ds(base, CH)], dst_v[slot],
                                  isem[slot]).wait()
            pltpu.make_async_copy(w_hbm.at[pl.ds(base, CH)], w_v[slot],
                                  isem[slot]).wait()

        def start_gather(slot, b):
            pltpu.async_copy(queue_hbm.at[src_v[slot]], rows[b], gsem[b])

        def wait_gather(b):
            pltpu.make_async_copy(queue_hbm.at[src_v[0]], rows[b],
                                  gsem[b]).wait()

        def drain_scatter(b):
            pltpu.make_async_copy(rows[b], acc.at[dst_v[0]],
                                  ssem[b]).wait()

        def zrow(i, carry):
            for kk in range(d_vregs):
                rows[0][i, pl.ds(kk * LANES, LANES)] = jnp.zeros(
                    (LANES,), jnp.float32)
            return carry
        lax.fori_loop(0, CH, zrow, 0)
        row0 = s * rows_per_tile
        for off, cnt in ((0, 112), (112, 112), (224, 112), (336, 112),
                         (448, 112), (560, 64)):
            pltpu.sync_copy(rows[0].at[pl.ds(0, cnt)],
                            acc.at[pl.ds(row0 + off, cnt)])

        @pl.when(s == N_SUBCORES - 1)
        def _zero_tail():
            pltpu.sync_copy(rows[0].at[pl.ds(0, tail_rows)],
                            acc.at[pl.ds(tail_start, tail_rows)])

        start_idx(0, 0)
        start_idx(1, 1)
        start_idx(2, 2)
        wait_idx(0, 0)
        start_gather(0, 0)
        wait_idx(1, 1)
        start_gather(1, 1)

        plsc.subcore_barrier()

        def ring(i, carry):
            g0 = UNROLL * i
            for u in range(UNROLL):
                g = g0 + u
                b = u % NBUF
                wait_gather(b)

                @plsc.parallel_loop(0, n_groups, unroll=2)
                def _scale(gr):
                    wvec = w_v[u][pl.ds(gr * LANES, LANES)]
                    for j in range(LANES):
                        wb = lax.gather(
                            wvec,
                            jnp.full((LANES, 1), j, jnp.int32),
                            lax.GatherDimensionNumbers(
                                offset_dims=(),
                                collapsed_slice_dims=(0,),
                                start_index_map=(0,)),
                            (1,),
                            mode=lax.GatherScatterMode.PROMISE_IN_BOUNDS)
                        re = rows[b].at[gr * LANES + j]
                        for kk in range(d_vregs):
                            seg = re[pl.ds(kk * LANES, LANES)]
                            re[pl.ds(kk * LANES, LANES)] = seg * wb

                if u == 0:
                    @pl.when(i > 0)
                    def _():
                        drain_scatter((u + 2) % NBUF)
                else:
                    drain_scatter((u + 2) % NBUF)

                def _gather_next():
                    wait_idx(g + 2, (u + 2) % NSLOT)
                    start_gather((u + 2) % NSLOT, (u + 2) % NBUF)
                if u >= UNROLL - 2:
                    @pl.when(g + 2 < n_chunks)
                    def _():
                        _gather_next()
                else:
                    _gather_next()

                pltpu.async_copy(rows[b], acc.at[dst_v[u]], ssem[b],
                                 add=True)

                if u >= UNROLL - 3:
                    @pl.when(g + 3 < n_chunks)
                    def _():
                        start_idx(g + 3, (u + 3) % NSLOT)
                else:
                    start_idx(g + 3, (u + 3) % NSLOT)
            return carry
        lax.fori_loop(0, n_iters, ring, 0)

        drain_scatter((n_chunks - 1) % NBUF)
        plsc.subcore_barrier()

        pltpu.sync_copy(acc.at[pl.ds(row0, rows_per_tile)],
                        out_hbm.at[c, pl.ds(row0, rows_per_tile)])

        @pl.when(s == N_SUBCORES - 1)
        def _copy_tail():
            pltpu.sync_copy(acc.at[pl.ds(tail_start, tail_rows)],
                            out_hbm.at[c, pl.ds(tail_start, tail_rows)])

    return k(queue, src, dst, w)


def _tc_combine(partials):
    _, n_nodes, d_feat = partials.shape
    blk = 1000

    def add_body(p_ref, o_ref):
        o_ref[...] = p_ref[0] + p_ref[1]

    return pl.pallas_call(
        add_body,
        grid=(n_nodes // blk,),
        in_specs=[pl.BlockSpec((N_CORES, blk, d_feat), lambda i: (0, i, 0))],
        out_specs=pl.BlockSpec((blk, d_feat), lambda i: (i, 0)),
        out_shape=jax.ShapeDtypeStruct((n_nodes, d_feat), jnp.float32),
    )(partials)


def kernel(queue, edge_index, edge_weight):
    n_nodes = queue.shape[0]
    src = edge_index[0].astype(jnp.int32)
    dst = edge_index[1].astype(jnp.int32)
    w = edge_weight[:, 0]

    n_edges = src.shape[0]
    quantum = NW * CH * UNROLL
    e_pad = -(-n_edges // quantum) * quantum
    n_chunks = e_pad // (NW * CH)
    pad = e_pad - n_edges
    if pad:
        pad_idx = jnp.arange(pad, dtype=jnp.int32) % n_nodes
        src = jnp.concatenate([src, pad_idx])
        dst = jnp.concatenate([dst, pad_idx])
        w = jnp.concatenate([w, jnp.zeros((pad,), jnp.float32)])

    partials = _sc_segment_sum(queue, src, dst, w, n_chunks)
    return _tc_combine(partials)
